# baseline (speedup 1.0000x reference)
import functools

import jax
import jax.numpy as jnp
from jax import lax
from jax.experimental import pallas as pl
from jax.experimental.pallas import tpu as pltpu
from jax.experimental.pallas import tpu_sc as plsc

N_HIDDEN = 128
EMB = 16
BATCH = 16384
N_TEACH = 100000
N_TEACH_PAD = 100352
STAGE = N_TEACH_PAD // 16
N_MAT = 1000
N_MAT_PAD = 1024
NW = 32
BPW = BATCH // NW
CHUNK = 128
NCH = BPW // CHUNK
BSC = 25600
BS = 8192


def _score_body(w_ref, tT_ref, mT_ref, ot_ref, om_ref):
    wt = w_ref[0, N_HIDDEN:N_HIDDEN + EMB].reshape(EMB, 1)
    wm = w_ref[0, N_HIDDEN + EMB:].reshape(EMB, 1)
    ot_ref[...] = jnp.sum(tT_ref[...] * wt, axis=0)
    om_ref[...] = jnp.sum(mT_ref[...] * wm, axis=0)


def _scores(tT, mT):
    del tT, mT
    grid = (N_TEACH_PAD + BSC - 1) // BSC
    return pl.pallas_call(
        _score_body,
        grid=(grid,),
        in_specs=[
            pl.BlockSpec((1, N_HIDDEN + 2 * EMB), lambda i: (0, 0)),
            pl.BlockSpec((EMB, BSC), lambda i: (0, i)),
            pl.BlockSpec((EMB, N_MAT_PAD), lambda i: (0, 0)),
        ],
        out_specs=(pl.BlockSpec((BSC,), lambda i: (i,)),
                   pl.BlockSpec((N_MAT_PAD,), lambda i: (0,))),
        out_shape=(jax.ShapeDtypeStruct((N_TEACH_PAD,), jnp.float32),
                   jax.ShapeDtypeStruct((N_MAT_PAD,), jnp.float32)),
    )


def _sc_gather_scores(score_t, score_m, tid, mid):
    mesh = plsc.VectorSubcoreMesh(core_axis_name="c", subcore_axis_name="s")

    @functools.partial(
        pl.kernel,
        mesh=mesh,
        compiler_params=pltpu.CompilerParams(use_tc_tiling_on_sc=False),
        out_type=(jax.ShapeDtypeStruct((BATCH,), jnp.float32),
                  jax.ShapeDtypeStruct((BATCH,), jnp.float32)),
        scratch_types=[
            pltpu.VMEM((BPW,), jnp.int32),
            pltpu.VMEM((BPW,), jnp.int32),
            pltpu.VMEM((BPW,), jnp.float32),
            pltpu.VMEM((BPW,), jnp.float32),
            pltpu.SemaphoreType.DMA,
        ],
    )
    def k(st_hbm, sm_hbm, tid_hbm, mid_hbm, yt_hbm, ym_hbm,
          tid_v, mid_v, tval_v, mval_v, sem):
        wid = lax.axis_index("s") * 2 + lax.axis_index("c")
        base = wid * BPW
        pltpu.sync_copy(tid_hbm.at[pl.ds(base, BPW)], tid_v)
        pltpu.sync_copy(mid_hbm.at[pl.ds(base, BPW)], mid_v)
        c1 = pltpu.async_copy(st_hbm.at[tid_v], tval_v, sem)
        c2 = pltpu.async_copy(sm_hbm.at[mid_v], mval_v, sem)
        c1.wait()
        c2.wait()
        pltpu.sync_copy(tval_v, yt_hbm.at[pl.ds(base, BPW)])
        pltpu.sync_copy(mval_v, ym_hbm.at[pl.ds(base, BPW)])

    return k(score_t, score_m, tid, mid)


def _head_body(b_ref, w_ref, h_ref, o_ref):
    whc = w_ref[0, :N_HIDDEN].reshape(N_HIDDEN, 1)
    yh = jax.lax.dot_general(h_ref[...], whc, (((1,), (0,)), ((), ())),
                             preferred_element_type=jnp.float32)
    o_ref[...] = yh.reshape(BS) + b_ref[0]


def _head(h, w, b):
    return pl.pallas_call(
        _head_body,
        grid=(BATCH // BS,),
        in_specs=[
            pl.BlockSpec(memory_space=pltpu.SMEM),
            pl.BlockSpec((1, N_HIDDEN + 2 * EMB), lambda i: (0, 0)),
            pl.BlockSpec((BS, N_HIDDEN), lambda i: (i, 0)),
        ],
        out_specs=pl.BlockSpec((BS,), lambda i: (i,)),
        out_shape=jax.ShapeDtypeStruct((BATCH,), jnp.float32),
    )(b, w, h)


def _add_body(a_ref, b_ref, c_ref, o_ref):
    o_ref[...] = a_ref[...] + b_ref[...] + c_ref[...]


def _final_add(y_h, y_t, y_m):
    return pl.pallas_call(
        _add_body,
        out_shape=jax.ShapeDtypeStruct((BATCH,), jnp.float32),
    )(y_h, y_t, y_m)


def kernel(h, teacher_id, materia_id, teacher_emb, materia_emb, W, b):
    tid = teacher_id.astype(jnp.int32)
    mid = materia_id.astype(jnp.int32)
    score_t, score_m = _scores(teacher_emb.T, materia_emb.T)(
        W, teacher_emb.T, materia_emb.T)
    y_t, y_m = _sc_gather_scores(score_t, score_m, tid, mid)
    y_h = _head(h, W, b)
    return _final_add(y_h, y_t, y_m)


# speedup vs baseline: 1.1447x; 1.1447x over previous
import functools

import jax
import jax.numpy as jnp
from jax import lax
from jax.experimental import pallas as pl
from jax.experimental.pallas import tpu as pltpu
from jax.experimental.pallas import tpu_sc as plsc

N_HIDDEN = 128
EMB = 16
BATCH = 16384
N_TEACH = 100000
N_TEACH_PAD = 100352
STAGE = N_TEACH_PAD // 16
N_MAT = 1000
N_MAT_PAD = 1024
NW = 32
BPW = BATCH // NW
CHUNK = 128
NCH = BPW // CHUNK
BSC = 51200
BS = 8192


def _score_body(w_ref, tT_ref, mT_ref, ot_ref, om_ref):
    wt = w_ref[0, N_HIDDEN:N_HIDDEN + EMB].reshape(EMB, 1)
    wm = w_ref[0, N_HIDDEN + EMB:].reshape(EMB, 1)
    ot_ref[...] = jnp.sum(tT_ref[...] * wt, axis=0)
    om_ref[...] = jnp.sum(mT_ref[...] * wm, axis=0)


def _scores(tT, mT):
    del tT, mT
    grid = (N_TEACH_PAD + BSC - 1) // BSC
    return pl.pallas_call(
        _score_body,
        grid=(grid,),
        in_specs=[
            pl.BlockSpec((1, N_HIDDEN + 2 * EMB), lambda i: (0, 0)),
            pl.BlockSpec((EMB, BSC), lambda i: (0, i)),
            pl.BlockSpec((EMB, N_MAT_PAD), lambda i: (0, 0)),
        ],
        out_specs=(pl.BlockSpec((BSC,), lambda i: (i,)),
                   pl.BlockSpec((N_MAT_PAD,), lambda i: (0,))),
        out_shape=(jax.ShapeDtypeStruct((N_TEACH_PAD,), jnp.float32),
                   jax.ShapeDtypeStruct((N_MAT_PAD,), jnp.float32)),
    )


def _sc_gather_scores(score_t, score_m, tid, mid):
    mesh = plsc.VectorSubcoreMesh(core_axis_name="c", subcore_axis_name="s")

    @functools.partial(
        pl.kernel,
        mesh=mesh,
        compiler_params=pltpu.CompilerParams(
            use_tc_tiling_on_sc=False, needs_layout_passes=False),
        out_type=jax.ShapeDtypeStruct((BATCH,), jnp.float32),
        scratch_types=[
            pltpu.VMEM((BPW,), jnp.int32),
            pltpu.VMEM((BPW,), jnp.int32),
            pltpu.VMEM((BPW,), jnp.float32),
            pltpu.VMEM((N_MAT_PAD,), jnp.float32),
            pltpu.VMEM((BPW,), jnp.float32),
            pltpu.VMEM_SHARED((N_TEACH_PAD,), jnp.float32),
            pltpu.SemaphoreType.DMA,
        ],
    )
    def k(st_hbm, sm_hbm, tid_hbm, mid_hbm, out_hbm,
          tid_v, mid_v, tval_v, sm_v, y_v, st_sh, sem):
        sid = lax.axis_index("s")
        wid = sid * 2 + lax.axis_index("c")
        base = wid * BPW
        pltpu.sync_copy(st_hbm.at[pl.ds(sid * STAGE, STAGE)],
                        st_sh.at[pl.ds(sid * STAGE, STAGE)])
        pltpu.sync_copy(tid_hbm.at[pl.ds(base, BPW)], tid_v)
        pltpu.sync_copy(mid_hbm.at[pl.ds(base, BPW)], mid_v)
        pltpu.sync_copy(sm_hbm, sm_v)
        plsc.subcore_barrier()
        pltpu.async_copy(st_sh.at[tid_v], tval_v, sem).wait()
        for g in range(BPW // 16):
            mval = plsc.load_gather(sm_v, [mid_v[pl.ds(g * 16, 16)]])
            y_v[pl.ds(g * 16, 16)] = tval_v[pl.ds(g * 16, 16)] + mval
        pltpu.sync_copy(y_v, out_hbm.at[pl.ds(base, BPW)])

    return k(score_t, score_m, tid, mid)


def _head_body(b_ref, w_ref, h_ref, o_ref):
    whc = w_ref[0, :N_HIDDEN].reshape(N_HIDDEN, 1)
    yh = jax.lax.dot_general(h_ref[...], whc, (((1,), (0,)), ((), ())),
                             preferred_element_type=jnp.float32)
    o_ref[...] = yh.reshape(BS) + b_ref[0]


def _head(h, w, b):
    return pl.pallas_call(
        _head_body,
        grid=(BATCH // BS,),
        in_specs=[
            pl.BlockSpec(memory_space=pltpu.SMEM),
            pl.BlockSpec((1, N_HIDDEN + 2 * EMB), lambda i: (0, 0)),
            pl.BlockSpec((BS, N_HIDDEN), lambda i: (i, 0)),
        ],
        out_specs=pl.BlockSpec((BS,), lambda i: (i,)),
        out_shape=jax.ShapeDtypeStruct((BATCH,), jnp.float32),
    )(b, w, h)


def _add_body(a_ref, b_ref, o_ref):
    o_ref[...] = a_ref[...] + b_ref[...]


def _final_add(y_h, y_tm):
    return pl.pallas_call(
        _add_body,
        out_shape=jax.ShapeDtypeStruct((BATCH,), jnp.float32),
    )(y_h, y_tm)


def kernel(h, teacher_id, materia_id, teacher_emb, materia_emb, W, b):
    tid = teacher_id.astype(jnp.int32)
    mid = materia_id.astype(jnp.int32)
    score_t, score_m = _scores(teacher_emb.T, materia_emb.T)(
        W, teacher_emb.T, materia_emb.T)
    y_tm = _sc_gather_scores(score_t, score_m, tid, mid)
    y_h = _head(h, W, b)
    return _final_add(y_h, y_tm)
